# baseline (speedup 1.0000x reference)
import functools

import jax
import jax.numpy as jnp
from jax import lax
from jax.experimental import pallas as pl
from jax.experimental.pallas import tpu as pltpu
from jax.experimental.pallas import tpu_sc as plsc

BATCH = 4096
HIST = 200
D = 64
DP = 128
NC = 2
NS = 16
NW = NC * NS
BPW = BATCH // NW
HPAIRS = HIST // 2
NSLOT = 3
NTRI = 33

_mesh = plsc.VectorSubcoreMesh(core_axis_name="c", subcore_axis_name="s")

LEN_V = 1000000
NBLK = 7813
LEN_VP = NBLK * 128
NT = 246


@functools.partial(
    pl.kernel,
    out_type=jax.ShapeDtypeStruct((LEN_VP, DP), jnp.float32),
    mesh=_mesh,
    scratch_types=[
        pltpu.VMEM((D, DP), jnp.float32),
        pltpu.VMEM((D, DP), jnp.float32),
        pltpu.VMEM((DP, DP), jnp.float32),
        pltpu.VMEM((DP, DP), jnp.float32),
        pltpu.SemaphoreType.DMA,
        pltpu.SemaphoreType.DMA,
        pltpu.SemaphoreType.DMA,
        pltpu.SemaphoreType.DMA,
    ],
    compiler_params=pltpu.CompilerParams(needs_layout_passes=False),
)
def _sc_format(ewT_hbm, tail_hbm, tab_hbm, in_v0, in_v1, out_v0, out_v1,
               i0s, i1s, o0s, o1s):
    isem = (i0s, i1s)
    osem = (o0s, o1s)
    in_v = (in_v0, in_v1)
    out_v = (out_v0, out_v1)
    wid = lax.axis_index("s") * NC + lax.axis_index("c")
    iota16 = lax.iota(jnp.int32, 16)

    def blk(t):
        return jnp.minimum(wid + NW * t, NBLK - 1)

    def start_in(t, s):
        k = blk(t)

        @pl.when(k < NBLK - 1)
        def _():
            pltpu.async_copy(
                ewT_hbm.at[:, pl.ds(pl.multiple_of(k * DP, DP), DP)],
                in_v[s],
                isem[s],
            )

        @pl.when(k == NBLK - 1)
        def _():
            pltpu.async_copy(tail_hbm, in_v[s], isem[s])

    def wait_in(s):
        pltpu.make_async_copy(
            ewT_hbm.at[:, pl.ds(0, DP)], in_v[s], isem[s]
        ).wait()

    def start_out(t, s):
        pltpu.async_copy(
            out_v[s],
            tab_hbm.at[pl.ds(pl.multiple_of(blk(t) * DP, DP), DP)],
            osem[s],
        )

    def wait_out(s):
        pltpu.make_async_copy(
            tab_hbm.at[pl.ds(0, DP)], out_v[s], osem[s]
        ).wait()

    def transpose_block(s):
        def cbody(c, _):
            csplat = jnp.full((16,), c, jnp.int32)
            for jb in range(D // 16):
                jvec = iota16 + (16 * jb)
                vals = plsc.load_gather(in_v[s], [jvec, csplat])
                plsc.store_scatter(out_v[s], [csplat, jvec], vals)
            return 0

        lax.fori_loop(0, DP, cbody, 0)

    start_in(0, 0)

    def body(q, _):
        for half in range(2):
            t = 2 * q + half
            s = half

            @pl.when(t + 1 < NT)
            def _():
                start_in(t + 1, 1 - s)

            wait_in(s)

            @pl.when(t >= 2)
            def _():
                wait_out(s)
            transpose_block(s)
            start_out(t, s)

        return 0

    lax.fori_loop(0, NT // 2, body, 0)
    wait_out(0)
    wait_out(1)


@functools.partial(
    pl.kernel,
    out_type=jax.ShapeDtypeStruct((BATCH, HIST, DP), jnp.float32),
    mesh=_mesh,
    scratch_types=[
        pltpu.VMEM((HIST, BPW), jnp.int32),
        pltpu.VMEM((3, BPW, 2, DP), jnp.float32),
        pltpu.SemaphoreType.DMA,
        pltpu.SemaphoreType.DMA,
        pltpu.SemaphoreType.DMA,
        pltpu.SemaphoreType.DMA,
        pltpu.SemaphoreType.DMA,
        pltpu.SemaphoreType.DMA,
    ],
)
def _sc_gather(idxT_hbm, tab_hbm, out_hbm, idx_v, rows_v,
               g0, g1, g2, o0, o1, o2):
    gsem = (g0, g1, g2)
    osem = (o0, o1, o2)
    wid = lax.axis_index("s") * NC + lax.axis_index("c")
    b0 = wid * BPW
    pltpu.sync_copy(idxT_hbm.at[:, pl.ds(b0, BPW)], idx_v)

    def start_pair(p, slot):
        pltpu.async_copy(
            tab_hbm.at[idx_v.at[2 * p]], rows_v.at[slot, :, 0, :], gsem[slot]
        )
        pltpu.async_copy(
            tab_hbm.at[idx_v.at[2 * p + 1]],
            rows_v.at[slot, :, 1, :],
            gsem[slot],
        )

    def wait_pair(slot):
        for hh in range(2):
            pltpu.make_async_copy(
                tab_hbm.at[pl.ds(0, BPW)],
                rows_v.at[slot, :, hh, :],
                gsem[slot],
            ).wait()

    def start_write(p, slot):
        pltpu.async_copy(
            rows_v.at[slot],
            out_hbm.at[pl.ds(b0, BPW), pl.ds(2 * p, 2), :],
            osem[slot],
        )

    def wait_write(slot):
        pltpu.make_async_copy(
            out_hbm.at[pl.ds(0, BPW), pl.ds(0, 2), :],
            rows_v.at[slot],
            osem[slot],
        ).wait()

    start_pair(0, 0)
    start_pair(1, 1)
    start_pair(2, 2)

    def body(q, _):
        for k in range(NSLOT):
            p = 3 * q + k
            wait_pair(k)
            start_write(p, k)
            ns = (k + 2) % NSLOT
            if k == 0:
                @pl.when(q >= 1)
                def _():
                    wait_write(ns)
                    start_pair(p + 2, ns)
            elif k == 2:
                @pl.when(q < NTRI - 1)
                def _():
                    wait_write(ns)
                    start_pair(p + 2, ns)
            else:
                wait_write(ns)
                start_pair(p + 2, ns)

        return 0

    lax.fori_loop(0, NTRI, body, 0)
    wait_pair(0)
    start_write(HPAIRS - 1, 0)
    wait_write(0)
    wait_write(1)
    wait_write(2)


def kernel(indices, embed_weight):
    idx_t = jnp.transpose(indices.astype(jnp.int32))
    ew_t = jnp.transpose(embed_weight)
    ntail = LEN_V - (NBLK - 1) * DP
    ew_tail = jnp.pad(ew_t[:, (NBLK - 1) * DP:], ((0, 0), (0, DP - ntail)))
    tab128 = _sc_format(ew_t, ew_tail)
    return _sc_gather(idx_t, tab128)[:, :, :D]


# speedup vs baseline: 1.9698x; 1.9698x over previous
import functools

import jax
import jax.numpy as jnp
from jax import lax
from jax.experimental import pallas as pl
from jax.experimental.pallas import tpu as pltpu
from jax.experimental.pallas import tpu_sc as plsc

BATCH = 4096
HIST = 200
D = 64
DP = 128
NC = 2
NS = 16
NW = NC * NS
BPW = BATCH // NW
HPAIRS = HIST // 2
NSLOT = 3
NTRI = 33

_mesh = plsc.VectorSubcoreMesh(core_axis_name="c", subcore_axis_name="s")


@functools.partial(
    pl.kernel,
    out_type=jax.ShapeDtypeStruct((BATCH, HIST, DP), jnp.float32),
    mesh=_mesh,
    scratch_types=[
        pltpu.VMEM((HIST, BPW), jnp.int32),
        pltpu.VMEM((3, BPW, 2, DP), jnp.float32),
        pltpu.SemaphoreType.DMA,
        pltpu.SemaphoreType.DMA,
        pltpu.SemaphoreType.DMA,
        pltpu.SemaphoreType.DMA,
        pltpu.SemaphoreType.DMA,
        pltpu.SemaphoreType.DMA,
    ],
)
def _sc_gather(idxT_hbm, tab_hbm, out_hbm, idx_v, rows_v,
               g0, g1, g2, o0, o1, o2):
    gsem = (g0, g1, g2)
    osem = (o0, o1, o2)
    wid = lax.axis_index("s") * NC + lax.axis_index("c")
    b0 = wid * BPW
    pltpu.sync_copy(idxT_hbm.at[:, pl.ds(b0, BPW)], idx_v)

    def start_pair(p, slot):
        pltpu.async_copy(
            tab_hbm.at[idx_v.at[2 * p]], rows_v.at[slot, :, 0, :], gsem[slot]
        )
        pltpu.async_copy(
            tab_hbm.at[idx_v.at[2 * p + 1]],
            rows_v.at[slot, :, 1, :],
            gsem[slot],
        )

    def wait_pair(slot):
        for hh in range(2):
            pltpu.make_async_copy(
                tab_hbm.at[pl.ds(0, BPW)],
                rows_v.at[slot, :, hh, :],
                gsem[slot],
            ).wait()

    def start_write(p, slot):
        pltpu.async_copy(
            rows_v.at[slot],
            out_hbm.at[pl.ds(b0, BPW), pl.ds(2 * p, 2), :],
            osem[slot],
        )

    def wait_write(slot):
        pltpu.make_async_copy(
            out_hbm.at[pl.ds(0, BPW), pl.ds(0, 2), :],
            rows_v.at[slot],
            osem[slot],
        ).wait()

    start_pair(0, 0)
    start_pair(1, 1)
    start_pair(2, 2)

    def body(q, _):
        for k in range(NSLOT):
            p = 3 * q + k
            wait_pair(k)
            start_write(p, k)
            ns = (k + 2) % NSLOT
            if k == 0:
                @pl.when(q >= 1)
                def _():
                    wait_write(ns)
                    start_pair(p + 2, ns)
            elif k == 2:
                @pl.when(q < NTRI - 1)
                def _():
                    wait_write(ns)
                    start_pair(p + 2, ns)
            else:
                wait_write(ns)
                start_pair(p + 2, ns)

        return 0

    lax.fori_loop(0, NTRI, body, 0)
    wait_pair(0)
    start_write(HPAIRS - 1, 0)
    wait_write(0)
    wait_write(1)
    wait_write(2)


def kernel(indices, embed_weight):
    idx_t = jnp.transpose(indices.astype(jnp.int32))
    tab128 = jnp.pad(embed_weight, ((0, 0), (0, DP - D)))
    return _sc_gather(idx_t, tab128)[:, :, :D]
